# baseline (speedup 1.0000x reference)
import functools

import jax
import jax.numpy as jnp
from jax import lax
from jax.experimental import pallas as pl
from jax.experimental.pallas import tpu as pltpu
from jax.experimental.pallas import tpu_sc as plsc

_ACT_DIM = 2048
_DICT = 32768
_K = 64
_B = 8

_BD_E = 2048

_NQ = 4
_SLICE = _DICT // _NQ
_NCH = _ACT_DIM // 128


def _sortable_key(z):
    bits = jax.lax.bitcast_convert_type(z, jnp.int32)
    return jnp.where(bits >= 0, bits, bits ^ jnp.int32(0x7FFFFFFF))


def _encode_kernel(x_ref, w_ref, b_ref, sz_ref, z_scr, key_scr):
    i = pl.program_id(0)
    nb = pl.num_programs(0)
    zblk = jax.lax.dot_general(
        x_ref[...], w_ref[...], (((1,), (1,)), ((), ())),
        preferred_element_type=jnp.float32) + b_ref[...]
    z_scr[:, pl.ds(i * _BD_E, _BD_E)] = zblk
    key_scr[:, pl.ds(i * _BD_E, _BD_E)] = _sortable_key(zblk)

    @pl.when(i == nb - 1)
    def _finish():
        key = key_scr[...]

        def body(_, carry):
            lo, hi = carry
            mid = (lo >> 1) + (hi >> 1) + (lo & hi & 1)
            cnt = jnp.sum((key >= mid).astype(jnp.int32), axis=1,
                          keepdims=True)
            ge = cnt >= _K
            return jnp.where(ge, mid, lo), jnp.where(ge, hi, mid)

        lo0 = jnp.full((_B, 1), jnp.iinfo(jnp.int32).min, jnp.int32)
        hi0 = jnp.full((_B, 1), jnp.iinfo(jnp.int32).max, jnp.int32)
        thr, _ = jax.lax.fori_loop(0, 32, body, (lo0, hi0))
        sz_ref[...] = jnp.where(key >= thr, z_scr[...], 0.0)


def _sc_decode_body(sz_hbm, wdec_hbm, bdec_hbm, out_hbm,
                    sz_v, idxs_v, vals_v, idxbase_v, idx_v, col_v,
                    acc_v, bdec_v, sum4_v, stage_sh, sem):
    c = lax.axis_index("c")
    s = lax.axis_index("s")
    b_loc = s // _NQ
    q = s % _NQ
    b = c * 4 + b_loc

    lane = lax.broadcasted_iota(jnp.int32, (16,), 0)
    zero16 = jnp.zeros((16,), jnp.float32)

    def vgather(x, idx):
        return lax.gather(
            x, idx[:, None],
            dimension_numbers=lax.GatherDimensionNumbers(
                offset_dims=(), collapsed_slice_dims=(0,),
                start_index_map=(0,)),
            slice_sizes=(1,),
            mode=lax.GatherScatterMode.PROMISE_IN_BOUNDS)

    pltpu.sync_copy(sz_hbm.at[pl.ds(b * _DICT + q * _SLICE, _SLICE)], sz_v)

    for m in range(128):
        idxbase_v[pl.ds(16 * m, 16)] = (lane + 16 * m) * _DICT

    for m in range(128):
        acc_v[pl.ds(16 * m, 16)] = zero16

    @pl.when(q == 0)
    def _init_bias():
        pltpu.sync_copy(bdec_hbm, bdec_v)
        for m in range(128):
            acc_v[pl.ds(16 * m, 16)] = bdec_v[pl.ds(16 * m, 16)]

    if True:
        ones16 = jnp.ones((16,), jnp.int32)
        zeros16 = jnp.zeros((16,), jnp.int32)
        sixteen16 = jnp.full((16,), 16, jnp.int32)

        def ext_body(t, off):
            v = sz_v[pl.ds(16 * t, 16)]
            m = v != 0.0
            mm0 = jnp.where(m, ones16, zeros16)
            anyv = mm0
            for sh in (1, 2, 4, 8):
                anyv = anyv | vgather(anyv, lane ^ sh)

            def do_comp(off_i):
                cum = mm0
                for sh in (1, 2, 4, 8):
                    g = vgather(cum, jnp.maximum(lane - sh, zeros16))
                    cum = cum + jnp.where(lane >= sh, g, zeros16)
                c = cum[15]
                pos = zeros16 - 1
                for sh in (8, 4, 2, 1):
                    cand = pos + sh
                    cv = vgather(cum, cand)
                    pos = jnp.where(cv < lane + 1, cand, pos)
                srcl = pos + 1
                vals_v[pl.ds(off_i, 16)] = vgather(v, srcl)
                idxs_v[pl.ds(off_i, 16)] = srcl + 16 * t
                return off_i + c

            return lax.cond(anyv[0] > 0, do_comp, lambda o: o, off)
        cnt = lax.fori_loop(0, _SLICE // 16, ext_body, jnp.int32(0))

    if True:
        def g_body(i, _):
            j = idxs_v[pl.ds(i, 16)][0] + q * _SLICE
            v = vals_v[pl.ds(i, 16)][0]
            for m in range(128):
                d = pl.ds(16 * m, 16)
                idx_v[d] = idxbase_v[d] + j
            pltpu.async_copy(wdec_hbm.at[idx_v], col_v, sem).wait()
            for m in range(128):
                d = pl.ds(16 * m, 16)
                acc_v[d] += v * col_v[d]
            return 0
        lax.fori_loop(0, cnt, g_body, 0)

    pltpu.sync_copy(acc_v, stage_sh.at[s])
    plsc.subcore_barrier()

    @pl.when(q == 0)
    def _reduce_row():
        pltpu.sync_copy(stage_sh.at[pl.ds(4 * b_loc, 4)], sum4_v)
        for m in range(128):
            d = pl.ds(16 * m, 16)
            acc_v[d] = (sum4_v[0, d] + sum4_v[1, d]
                        + sum4_v[2, d] + sum4_v[3, d])
        pltpu.sync_copy(acc_v, out_hbm.at[b])


_sc_decode = functools.partial(
    pl.kernel,
    out_type=jax.ShapeDtypeStruct((_B, _ACT_DIM), jnp.float32),
    mesh=plsc.VectorSubcoreMesh(core_axis_name="c", subcore_axis_name="s"),
    scratch_types=[
        pltpu.VMEM((_SLICE,), jnp.float32),
        pltpu.VMEM((_SLICE + 16,), jnp.int32),
        pltpu.VMEM((_SLICE + 16,), jnp.float32),
        pltpu.VMEM((_ACT_DIM,), jnp.int32),
        pltpu.VMEM((_ACT_DIM,), jnp.int32),
        pltpu.VMEM((_ACT_DIM,), jnp.float32),
        pltpu.VMEM((_ACT_DIM,), jnp.float32),
        pltpu.VMEM((_ACT_DIM,), jnp.float32),
        pltpu.VMEM((4, _ACT_DIM), jnp.float32),
        pltpu.VMEM_SHARED((16, _ACT_DIM), jnp.float32),
        pltpu.SemaphoreType.DMA,
    ],
)(_sc_decode_body)


@jax.jit
def kernel(x, W_enc, b_enc, W_dec, b_dec):
    b_enc2 = b_enc.reshape(1, _DICT)

    nb_e = _DICT // _BD_E
    sparse_z = pl.pallas_call(
        _encode_kernel,
        grid=(nb_e,),
        in_specs=[
            pl.BlockSpec((_B, _ACT_DIM), lambda i: (0, 0)),
            pl.BlockSpec((_BD_E, _ACT_DIM), lambda i: (i, 0)),
            pl.BlockSpec((1, _BD_E), lambda i: (0, i)),
        ],
        out_specs=pl.BlockSpec((_B, _DICT), lambda i: (0, 0)),
        out_shape=jax.ShapeDtypeStruct((_B, _DICT), jnp.float32),
        scratch_shapes=[
            pltpu.VMEM((_B, _DICT), jnp.float32),
            pltpu.VMEM((_B, _DICT), jnp.int32),
        ],
    )(x, W_enc, b_enc2)

    x_hat = _sc_decode(sparse_z.reshape(-1), W_dec.reshape(-1), b_dec)
    return (x_hat, sparse_z)


# speedup vs baseline: 2.0963x; 2.0963x over previous
import jax
import jax.numpy as jnp
from jax.experimental import pallas as pl
from jax.experimental.pallas import tpu as pltpu

_ACT_DIM = 2048
_DICT = 32768
_K = 64
_B = 8

_BD = 1024
_NB = _DICT // _BD


def _sortable_key(z):
    bits = jax.lax.bitcast_convert_type(z, jnp.int32)
    return jnp.where(bits >= 0, bits, bits ^ jnp.int32(0x7FFFFFFF))


def _fused_kernel(x_ref, we_ref, be_ref, wd_ref, bd_ref, sz_ref, out_ref,
                  z_scr, key_scr, thr_scr, acc):
    p = pl.program_id(0)
    i = pl.program_id(1)

    @pl.when(p == 0)
    def _encode():
        zblk = jax.lax.dot_general(
            x_ref[...], we_ref[...], (((1,), (1,)), ((), ())),
            preferred_element_type=jnp.float32) + be_ref[...]
        z_scr[:, pl.ds(i * _BD, _BD)] = zblk
        key_scr[:, pl.ds(i * _BD, _BD)] = _sortable_key(zblk)

        @pl.when(i == _NB - 1)
        def _finish():
            key = key_scr[...]

            def body(_, carry):
                lo, hi = carry
                mid = (lo >> 1) + (hi >> 1) + (lo & hi & 1)
                cnt = jnp.sum((key >= mid).astype(jnp.int32), axis=1,
                              keepdims=True)
                ge = cnt >= _K
                return jnp.where(ge, mid, lo), jnp.where(ge, hi, mid)

            lo0 = jnp.full((_B, 1), jnp.iinfo(jnp.int32).min, jnp.int32)
            hi0 = jnp.full((_B, 1), jnp.iinfo(jnp.int32).max, jnp.int32)
            thr, _ = jax.lax.fori_loop(0, 32, body, (lo0, hi0))
            thr_scr[...] = jnp.broadcast_to(thr, (_B, 128))
            sz_ref[...] = jnp.where(key >= thr, z_scr[...], 0.0)

    @pl.when(p == 1)
    def _decode():
        @pl.when(i == 0)
        def _init():
            acc[...] = jnp.zeros_like(acc)

        d = pl.ds(i * _BD, _BD)
        szblk = jnp.where(key_scr[:, d] >= thr_scr[:, :1], z_scr[:, d], 0.0)
        acc[...] += jax.lax.dot_general(
            szblk, wd_ref[...], (((1,), (1,)), ((), ())),
            preferred_element_type=jnp.float32)

        @pl.when(i == _NB - 1)
        def _finish():
            out_ref[...] = acc[...] + bd_ref[...]


@jax.jit
def kernel(x, W_enc, b_enc, W_dec, b_dec):
    b_enc2 = b_enc.reshape(1, _DICT)
    b_dec2 = b_dec.reshape(1, _ACT_DIM)
    nb = _NB

    sparse_z, x_hat = pl.pallas_call(
        _fused_kernel,
        grid=(2, nb),
        in_specs=[
            pl.BlockSpec((_B, _ACT_DIM), lambda p, i: (0, 0)),
            pl.BlockSpec((_BD, _ACT_DIM),
                         lambda p, i: (jnp.where(p == 0, i, nb - 1), 0)),
            pl.BlockSpec((1, _BD),
                         lambda p, i: (0, jnp.where(p == 0, i, nb - 1))),
            pl.BlockSpec((_ACT_DIM, _BD),
                         lambda p, i: (0, jnp.where(p == 1, i, 0))),
            pl.BlockSpec((1, _ACT_DIM), lambda p, i: (0, 0)),
        ],
        out_specs=[
            pl.BlockSpec((_B, _DICT), lambda p, i: (0, 0)),
            pl.BlockSpec((_B, _ACT_DIM), lambda p, i: (0, 0)),
        ],
        out_shape=[
            jax.ShapeDtypeStruct((_B, _DICT), jnp.float32),
            jax.ShapeDtypeStruct((_B, _ACT_DIM), jnp.float32),
        ],
        scratch_shapes=[
            pltpu.VMEM((_B, _DICT), jnp.float32),
            pltpu.VMEM((_B, _DICT), jnp.int32),
            pltpu.VMEM((_B, 128), jnp.int32),
            pltpu.VMEM((_B, _ACT_DIM), jnp.float32),
        ],
        compiler_params=pltpu.CompilerParams(
            dimension_semantics=("arbitrary", "arbitrary")),
    )(x, W_enc, b_enc2, W_dec, b_dec2)

    return (x_hat, sparse_z)
